# Initial kernel scaffold; baseline (speedup 1.0000x reference)
import jax
import jax.numpy as jnp
from jax.experimental import pallas as pl


def kernel(x, attention, edge_index, W1, b1, W2, b2):
    raise NotImplementedError("write your pallas kernel here")


# speedup vs baseline: 3.6436x; 3.6436x over previous
import functools

import jax
import jax.numpy as jnp
from jax import lax
from jax.experimental import pallas as pl
from jax.experimental.pallas import tpu as pltpu
from jax.experimental.pallas import tpu_sc as plsc

_N = 10000
_E = 320000
_D = 128
_DC = 64
_HOPS = 3
_EPS = 1e-06

_NC = 2
_NS = 16

_CHUNK = 512
_BATCH = 128
_EDGES_PER_TILE = 20480
_E_PAD = _EDGES_PER_TILE * _NS
_N_CHUNKS = _EDGES_PER_TILE // _CHUNK

_ROWS_LO = 640
_ROWS_HI = 400


def _hop_body(cur_hbm, src_hbm, dst_hbm, att_hbm, out_hbm,
              src_v, dst_v, att_v, rows_v, zero_v, acc_sh,
              sem_g0, sem_g1, sem_s0, sem_s1):
    c = lax.axis_index("c")
    s = lax.axis_index("s")
    sem_g = (sem_g0, sem_g1)
    sem_s = (sem_s0, sem_s1)

    def _zrow(r, _):
        for d in range(_DC // 16):
            zero_v[r, pl.ds(d * 16, 16)] = jnp.zeros((16,), jnp.float32)
        return 0
    lax.fori_loop(0, 128, _zrow, 0)

    @pl.when(s < 15)
    def _():
        def _zb(j, _):
            pltpu.sync_copy(zero_v, acc_sh.at[pl.ds(s * _ROWS_LO + j * 128, 128)])
            return 0
        lax.fori_loop(0, _ROWS_LO // 128, _zb, 0)

    @pl.when(s == 15)
    def _():
        def _zb(j, _):
            pltpu.sync_copy(zero_v, acc_sh.at[pl.ds(9600 + j * 128, 128)])
            return 0
        lax.fori_loop(0, 3, _zb, 0)
        pltpu.sync_copy(zero_v.at[pl.ds(0, 16)], acc_sh.at[pl.ds(9984, 16)])

    plsc.subcore_barrier()

    _NB = _CHUNK // _BATCH

    def _load_idx(i, b):
        base_row = s * (_EDGES_PER_TILE // _BATCH) + i * _NB
        pltpu.sync_copy(src_hbm.at[pl.ds(base_row, _NB)], src_v.at[b])
        pltpu.sync_copy(dst_hbm.at[pl.ds(base_row, _NB)], dst_v.at[b])
        pltpu.sync_copy(
            att_hbm.at[pl.ds(s * _EDGES_PER_TILE + i * _CHUNK, _CHUNK)],
            att_v.at[b])

    def _start_gathers(b):
        for j in range(_NB):
            pltpu.async_copy(cur_hbm.at[c].at[src_v.at[b, j]],
                             rows_v.at[b, pl.ds(j * _BATCH, _BATCH)], sem_g[b])

    def _wait_gathers(b):
        for j in range(_NB):
            pltpu.make_async_copy(
                cur_hbm.at[c].at[src_v.at[b, j]],
                rows_v.at[b, pl.ds(j * _BATCH, _BATCH)], sem_g[b]).wait()

    def _start_scatters(b):
        for j in range(_NB):
            pltpu.async_copy(rows_v.at[b, pl.ds(j * _BATCH, _BATCH)],
                             acc_sh.at[dst_v.at[b, j]], sem_s[b], add=True)

    def _wait_scatters(b):
        for j in range(_NB):
            pltpu.make_async_copy(
                rows_v.at[b, pl.ds(j * _BATCH, _BATCH)],
                acc_sh.at[dst_v.at[b, j]], sem_s[b]).wait()

    def _scale(b):
        def _group(g, _):
            att16 = att_v[b, pl.ds(g * 16, 16)]
            for l in range(16):
                a = att16[l]
                e = g * 16 + l
                for d in range(_DC // 16):
                    rows_v[b, e, pl.ds(d * 16, 16)] = (
                        rows_v[b, e, pl.ds(d * 16, 16)] * a)
            return 0
        lax.fori_loop(0, _CHUNK // 16, _group, 0)

    _load_idx(0, 0)
    _start_gathers(0)

    def _body(io, _):
        i0 = 2 * io
        i1 = i0 + 1

        @pl.when(io > 0)
        def _():
            _wait_scatters(1)
        _load_idx(i1, 1)
        _start_gathers(1)

        _wait_gathers(0)
        _scale(0)
        _start_scatters(0)

        @pl.when(io < _N_CHUNKS // 2 - 1)
        def _():
            _wait_scatters(0)
            _load_idx(i0 + 2, 0)
            _start_gathers(0)

        _wait_gathers(1)
        _scale(1)
        _start_scatters(1)
        return 0
    lax.fori_loop(0, _N_CHUNKS // 2, _body, 0)

    _wait_scatters(0)
    _wait_scatters(1)

    plsc.subcore_barrier()

    @pl.when(s < 15)
    def _():
        pltpu.sync_copy(acc_sh.at[pl.ds(s * _ROWS_LO, _ROWS_LO)],
                        out_hbm.at[c, pl.ds(s * _ROWS_LO, _ROWS_LO)])

    @pl.when(s == 15)
    def _():
        pltpu.sync_copy(acc_sh.at[pl.ds(9600, _ROWS_HI)],
                        out_hbm.at[c, pl.ds(9600, _ROWS_HI)])


_hop_kernel = functools.partial(
    pl.kernel,
    out_type=jax.ShapeDtypeStruct((_NC, _N, _DC), jnp.float32),
    mesh=plsc.VectorSubcoreMesh(core_axis_name="c", subcore_axis_name="s"),
    scratch_types=[
        pltpu.VMEM((2, _CHUNK // _BATCH, _BATCH), jnp.int32),
        pltpu.VMEM((2, _CHUNK // _BATCH, _BATCH), jnp.int32),
        pltpu.VMEM((2, _CHUNK), jnp.float32),
        pltpu.VMEM((2, _CHUNK, _DC), jnp.float32),
        pltpu.VMEM((128, _DC), jnp.float32),
        pltpu.VMEM_SHARED((_N, _DC), jnp.float32),
        pltpu.SemaphoreType.DMA,
        pltpu.SemaphoreType.DMA,
        pltpu.SemaphoreType.DMA,
        pltpu.SemaphoreType.DMA,
    ],
    compiler_params=pltpu.CompilerParams(use_tc_tiling_on_sc=False),
)(_hop_body)


def _alpha_body(x_ref, w1_ref, b1_ref, w2t_ref, b2_ref, o_ref):
    g = jnp.mean(x_ref[...], axis=0, keepdims=True)
    h = jnp.tanh(jnp.dot(g, w1_ref[...],
                         preferred_element_type=jnp.float32) + b1_ref[...])
    logit = jnp.sum(h * w2t_ref[...]) + b2_ref[0, 0]
    a = jax.nn.sigmoid(logit)
    o_ref[...] = jnp.full((1, 1), jnp.clip(a, _EPS, 1.0 - _EPS), jnp.float32)


_alpha_kernel = pl.pallas_call(
    _alpha_body,
    out_shape=jax.ShapeDtypeStruct((1, 1), jnp.float32),
)


_BLK = 1000


def _combine_mid_body(alpha_ref, x_ref, p_ref, o_ref):
    a = alpha_ref[0, 0]
    o_ref[0] = jnp.clip(x_ref[:, :_DC] * a + p_ref[0] * (1.0 - a),
                        _EPS, 1.0 / _EPS)
    o_ref[1] = jnp.clip(x_ref[:, _DC:] * a + p_ref[1] * (1.0 - a),
                        _EPS, 1.0 / _EPS)


_combine_mid_kernel = pl.pallas_call(
    _combine_mid_body,
    grid=(_N // _BLK,),
    in_specs=[
        pl.BlockSpec((1, 1), lambda i: (0, 0)),
        pl.BlockSpec((_BLK, _D), lambda i: (i, 0)),
        pl.BlockSpec((_NC, _BLK, _DC), lambda i: (0, i, 0)),
    ],
    out_specs=pl.BlockSpec((_NC, _BLK, _DC), lambda i: (0, i, 0)),
    out_shape=jax.ShapeDtypeStruct((_NC, _N, _DC), jnp.float32),
)


def _combine_final_body(alpha_ref, x_ref, p_ref, o_ref):
    a = alpha_ref[0, 0]
    agg = jnp.concatenate([p_ref[0], p_ref[1]], axis=1)
    o_ref[...] = jnp.clip(x_ref[...] * a + agg * (1.0 - a), _EPS, 1.0 / _EPS)


_combine_final_kernel = pl.pallas_call(
    _combine_final_body,
    grid=(_N // _BLK,),
    in_specs=[
        pl.BlockSpec((1, 1), lambda i: (0, 0)),
        pl.BlockSpec((_BLK, _D), lambda i: (i, 0)),
        pl.BlockSpec((_NC, _BLK, _DC), lambda i: (0, i, 0)),
    ],
    out_specs=pl.BlockSpec((_BLK, _D), lambda i: (i, 0)),
    out_shape=jax.ShapeDtypeStruct((_N, _D), jnp.float32),
)


def kernel(x, attention, edge_index, W1, b1, W2, b2):
    pad = _E_PAD - _E
    src = jnp.concatenate([edge_index[0], jnp.zeros((pad,), jnp.int32)])
    dst = jnp.concatenate([edge_index[1], jnp.zeros((pad,), jnp.int32)])
    att = jnp.concatenate([attention, jnp.zeros((pad,), jnp.float32)])
    src2d = src.reshape(_E_PAD // _BATCH, _BATCH)
    dst2d = dst.reshape(_E_PAD // _BATCH, _BATCH)

    alpha = _alpha_kernel(x, W1, b1.reshape(1, _D), W2.reshape(1, _D),
                          b2.reshape(1, 1))

    cur = jnp.swapaxes(x.reshape(_N, _NC, _DC), 0, 1)

    for hop in range(_HOPS):
        parts = _hop_kernel(cur, src2d, dst2d, att)
        if hop < _HOPS - 1:
            cur = _combine_mid_kernel(alpha, x, parts)
        else:
            return _combine_final_kernel(alpha, x, parts)
